# Initial kernel scaffold; baseline (speedup 1.0000x reference)
import jax
import jax.numpy as jnp
from jax.experimental import pallas as pl


def kernel(x, edge_index, edge_weight, W1, b1, g1, be1, W2, b2, g2, be2, Wl, bl):
    raise NotImplementedError("write your pallas kernel here")


# speedup vs baseline: 6.1122x; 6.1122x over previous
import functools

import jax
import jax.numpy as jnp
from jax import lax
from jax.experimental import pallas as pl
from jax.experimental.pallas import tpu as pltpu
from jax.experimental.pallas import tpu_sc as plsc

N = 10000
E = 320000
D_IN = 128
H1 = 512
H2 = 300
H2P = 320
D_OUT = 40
EPS = 1e-5

CH = 80
TILES = 16
CORES = 2
RPT = 640
RLAST = N - RPT

BLK = 400
GRID = N // BLK


def _sc_degree(col, w2, zeros):
    epw = E // (CORES * TILES)
    nch = epw // CH
    mesh = plsc.VectorSubcoreMesh(core_axis_name="c", subcore_axis_name="s")

    @functools.partial(
        pl.kernel,
        out_type=jax.ShapeDtypeStruct((2 * N, 1), jnp.float32),
        mesh=mesh,
        scratch_types=[
            pltpu.VMEM((CH,), jnp.int32),
            pltpu.VMEM((CH, 1), jnp.float32),
            pltpu.VMEM_SHARED((N, 1), jnp.float32),
        ],
    )
    def k(col_h, w_h, z_h, out_h, coli, wv, acc):
        c = lax.axis_index("c")
        s = lax.axis_index("s")
        rbase = jnp.minimum(s * RPT, RLAST)
        pltpu.sync_copy(z_h.at[pl.ds(rbase, RPT)], acc.at[pl.ds(rbase, RPT)])
        plsc.subcore_barrier()
        ebase = (c * TILES + s) * epw

        def body(i, carry):
            off = ebase + i * CH
            pltpu.sync_copy(col_h.at[pl.ds(off, CH)], coli)
            pltpu.sync_copy(w_h.at[pl.ds(off, CH)], wv)
            pltpu.sync_copy(wv, acc.at[coli], add=True)
            return carry

        lax.fori_loop(0, nch, body, 0)
        plsc.subcore_barrier()
        pltpu.sync_copy(acc.at[pl.ds(rbase, RPT)],
                        out_h.at[pl.ds(c * N + rbase, RPT)])

    return k(col, w2, zeros)


def _sc_scatter(src, rowpc, col, wrep, dh):
    ept = E // TILES
    nch = ept // CH
    mesh = plsc.VectorSubcoreMesh(core_axis_name="c", subcore_axis_name="s")

    @functools.partial(
        pl.kernel,
        out_type=jax.ShapeDtypeStruct((2 * N, dh), jnp.float32),
        mesh=mesh,
        compiler_params=pltpu.CompilerParams(use_tc_tiling_on_sc=False),
        scratch_types=[
            pltpu.VMEM((CH,), jnp.int32),
            pltpu.VMEM((CH,), jnp.int32),
            pltpu.VMEM((CH * 16,), jnp.float32),
            pltpu.VMEM((CH, dh), jnp.float32),
            pltpu.VMEM_SHARED((N, dh), jnp.float32),
            pltpu.SemaphoreType.DMA,
        ],
    )
    def k(src_h, row_h, col_h, w_h, out_h, rowi, coli, wv, rows, acc, sem):
        c = lax.axis_index("c")
        s = lax.axis_index("s")
        rbase = jnp.minimum(s * RPT, RLAST)
        cn = c * N
        pltpu.sync_copy(src_h.at[pl.ds(cn + rbase, RPT)],
                        acc.at[pl.ds(rbase, RPT)])
        plsc.subcore_barrier()
        ebase = s * ept
        ce = c * E

        def body(i, carry):
            off = ebase + i * CH
            pltpu.sync_copy(row_h.at[pl.ds(ce + off, CH)], rowi)
            pltpu.sync_copy(col_h.at[pl.ds(off, CH)], coli)
            pltpu.sync_copy(w_h.at[pl.ds(off * 16, CH * 16)], wv)
            pltpu.async_copy(src_h.at[rowi], rows, sem).wait()

            def scale(r, cy):
                wb = wv[pl.ds(r * 16, 16)]
                for j in range(dh // 16):
                    rows[r, pl.ds(16 * j, 16)] = rows[r, pl.ds(16 * j, 16)] * wb
                return cy

            lax.fori_loop(0, CH, scale, 0)
            pltpu.sync_copy(rows, acc.at[coli], add=True)
            return carry

        lax.fori_loop(0, nch, body, 0)
        plsc.subcore_barrier()
        pltpu.sync_copy(acc.at[pl.ds(rbase, RPT)],
                        out_h.at[pl.ds(cn + rbase, RPT)])

    return k(src, rowpc, col, wrep)


def _tc_prep(x, degp):
    def body(x_ref, d0_ref, d1_ref, xs_ref, dis_ref):
        deg = d0_ref[...] + d1_ref[...] + 1.0
        dis = lax.rsqrt(deg)
        xs = x_ref[...] * dis
        xs_ref[0] = xs[:, : D_IN // 2]
        xs_ref[1] = xs[:, D_IN // 2:]
        dis_ref[...] = dis

    return pl.pallas_call(
        body,
        grid=(GRID,),
        in_specs=[
            pl.BlockSpec((BLK, D_IN), lambda i: (i, 0)),
            pl.BlockSpec((BLK, 1), lambda i: (i, 0)),
            pl.BlockSpec((BLK, 1), lambda i: (i, 0)),
        ],
        out_specs=[
            pl.BlockSpec((2, BLK, D_IN // 2), lambda i: (0, i, 0)),
            pl.BlockSpec((BLK, 1), lambda i: (i, 0)),
        ],
        out_shape=[
            jax.ShapeDtypeStruct((2, N, D_IN // 2), jnp.float32),
            jax.ShapeDtypeStruct((N, 1), jnp.float32),
        ],
    )(x, degp[:N], degp[N:])


def _tc_mm1(s1, dis, W1, b1):
    def body(s1_ref, dis_ref, w_ref, b_ref, y_ref):
        a = jnp.concatenate([s1_ref[0], s1_ref[1]], axis=1) * dis_ref[...]
        y_ref[...] = jnp.dot(a, w_ref[...],
                             preferred_element_type=jnp.float32) + b_ref[...]

    return pl.pallas_call(
        body,
        grid=(GRID,),
        in_specs=[
            pl.BlockSpec((2, BLK, D_IN // 2), lambda i: (0, i, 0)),
            pl.BlockSpec((BLK, 1), lambda i: (i, 0)),
            pl.BlockSpec((D_IN, H1), lambda i: (0, 0)),
            pl.BlockSpec((1, H1), lambda i: (0, 0)),
        ],
        out_specs=pl.BlockSpec((BLK, H1), lambda i: (i, 0)),
        out_shape=jax.ShapeDtypeStruct((N, H1), jnp.float32),
    )(s1, dis, W1, b1.reshape(1, H1))


def _tc_stats(y, h):
    def body(y_ref, mean_ref, rstd_ref, acc_ref):
        i = pl.program_id(0)

        @pl.when(i == 0)
        def _():
            acc_ref[...] = jnp.zeros_like(acc_ref)

        blk = y_ref[...]
        acc_ref[0:1] += jnp.sum(blk, axis=0, keepdims=True)
        acc_ref[1:2] += jnp.sum(blk * blk, axis=0, keepdims=True)

        @pl.when(i == GRID - 1)
        def _():
            m = acc_ref[0:1] / N
            v = acc_ref[1:2] / N - m * m
            mean_ref[...] = m
            rstd_ref[...] = lax.rsqrt(v + EPS)

    return pl.pallas_call(
        body,
        grid=(GRID,),
        in_specs=[pl.BlockSpec((BLK, h), lambda i: (i, 0))],
        out_specs=[
            pl.BlockSpec((1, h), lambda i: (0, 0)),
            pl.BlockSpec((1, h), lambda i: (0, 0)),
        ],
        out_shape=[jax.ShapeDtypeStruct((1, h), jnp.float32)] * 2,
        scratch_shapes=[pltpu.VMEM((2, h), jnp.float32)],
    )(y)


def _tc_mm2(y, mean, rstd, g1, be1, W2p, dis):
    def body(y_ref, m_ref, r_ref, g_ref, be_ref, w_ref, dis_ref, ts_ref):
        hh = jnp.maximum(
            g_ref[...] * (y_ref[...] - m_ref[...]) * r_ref[...] + be_ref[...],
            0.0)
        t = jnp.dot(hh, w_ref[...],
                    preferred_element_type=jnp.float32) * dis_ref[...]
        ts_ref[0] = t[:, : H2P // 2]
        ts_ref[1] = t[:, H2P // 2:]

    return pl.pallas_call(
        body,
        grid=(GRID,),
        in_specs=[
            pl.BlockSpec((BLK, H1), lambda i: (i, 0)),
            pl.BlockSpec((1, H1), lambda i: (0, 0)),
            pl.BlockSpec((1, H1), lambda i: (0, 0)),
            pl.BlockSpec((1, H1), lambda i: (0, 0)),
            pl.BlockSpec((1, H1), lambda i: (0, 0)),
            pl.BlockSpec((H1, H2P), lambda i: (0, 0)),
            pl.BlockSpec((BLK, 1), lambda i: (i, 0)),
        ],
        out_specs=pl.BlockSpec((2, BLK, H2P // 2), lambda i: (0, i, 0)),
        out_shape=jax.ShapeDtypeStruct((2, N, H2P // 2), jnp.float32),
    )(y, mean, rstd, g1.reshape(1, H1), be1.reshape(1, H1), W2p, dis)


def _tc_z(s2, dis, b2p):
    def body(s2_ref, dis_ref, b_ref, z_ref):
        z_ref[...] = (jnp.concatenate([s2_ref[0], s2_ref[1]], axis=1)
                      * dis_ref[...] + b_ref[...])

    return pl.pallas_call(
        body,
        grid=(GRID,),
        in_specs=[
            pl.BlockSpec((2, BLK, H2P // 2), lambda i: (0, i, 0)),
            pl.BlockSpec((BLK, 1), lambda i: (i, 0)),
            pl.BlockSpec((1, H2P), lambda i: (0, 0)),
        ],
        out_specs=pl.BlockSpec((BLK, H2P), lambda i: (i, 0)),
        out_shape=jax.ShapeDtypeStruct((N, H2P), jnp.float32),
    )(s2, dis, b2p)


def _tc_out(z, mean, rstd, g2p, be2p, Wlp, bl):
    def body(z_ref, m_ref, r_ref, g_ref, be_ref, w_ref, bl_ref, o_ref):
        hh = jnp.maximum(
            g_ref[...] * (z_ref[...] - m_ref[...]) * r_ref[...] + be_ref[...],
            0.0)
        lg = jnp.maximum(
            jnp.dot(hh, w_ref[...], preferred_element_type=jnp.float32)
            + bl_ref[...], 0.0)
        mx = jnp.max(lg, axis=1, keepdims=True)
        lse = jnp.log(jnp.sum(jnp.exp(lg - mx), axis=1, keepdims=True)) + mx
        o_ref[...] = lg - lse

    return pl.pallas_call(
        body,
        grid=(GRID,),
        in_specs=[
            pl.BlockSpec((BLK, H2P), lambda i: (i, 0)),
            pl.BlockSpec((1, H2P), lambda i: (0, 0)),
            pl.BlockSpec((1, H2P), lambda i: (0, 0)),
            pl.BlockSpec((1, H2P), lambda i: (0, 0)),
            pl.BlockSpec((1, H2P), lambda i: (0, 0)),
            pl.BlockSpec((H2P, D_OUT), lambda i: (0, 0)),
            pl.BlockSpec((1, D_OUT), lambda i: (0, 0)),
        ],
        out_specs=pl.BlockSpec((BLK, D_OUT), lambda i: (i, 0)),
        out_shape=jax.ShapeDtypeStruct((N, D_OUT), jnp.float32),
    )(z, mean, rstd, g2p, be2p, Wlp, bl.reshape(1, D_OUT))


def kernel(x, edge_index, edge_weight, W1, b1, g1, be1, W2, b2, g2, be2,
           Wl, bl):
    ei = edge_index.astype(jnp.int32)
    row = ei[0]
    col = ei[1]
    w = edge_weight.astype(jnp.float32)

    rowpc = jnp.concatenate([row, row + N])
    wrep = jnp.broadcast_to(w[:, None], (E, 16)).reshape(16 * E)

    degp = _sc_degree(col, w.reshape(E, 1), jnp.zeros((N, 1), jnp.float32))
    xs, dis = _tc_prep(x, degp)

    s1 = _sc_scatter(xs.reshape(2 * N, D_IN // 2), rowpc, col, wrep,
                     D_IN // 2)
    y = _tc_mm1(s1.reshape(2, N, D_IN // 2), dis, W1, b1)
    m1, r1 = _tc_stats(y, H1)

    W2p = jnp.pad(W2, ((0, 0), (0, H2P - H2)))
    ts = _tc_mm2(y, m1, r1, g1, be1, W2p, dis)

    s2 = _sc_scatter(ts.reshape(2 * N, H2P // 2), rowpc, col, wrep, H2P // 2)
    b2p = jnp.pad(b2, (0, H2P - H2)).reshape(1, H2P)
    z = _tc_z(s2.reshape(2, N, H2P // 2), dis, b2p)
    m2, r2 = _tc_stats(z, H2P)

    g2p = jnp.pad(g2, (0, H2P - H2)).reshape(1, H2P)
    be2p = jnp.pad(be2, (0, H2P - H2)).reshape(1, H2P)
    Wlp = jnp.pad(Wl, ((0, H2P - H2), (0, 0)))
    return _tc_out(z, m2, r2, g2p, be2p, Wlp, bl)


# speedup vs baseline: 6.1301x; 1.0029x over previous
import functools

import jax
import jax.numpy as jnp
from jax import lax
from jax.experimental import pallas as pl
from jax.experimental.pallas import tpu as pltpu
from jax.experimental.pallas import tpu_sc as plsc

N = 10000
E = 320000
D_IN = 128
H1 = 512
H2 = 300
H2P = 320
D_OUT = 40
EPS = 1e-5

CH = 80
TILES = 16
CORES = 2
RPT = 640
RLAST = N - RPT

BLK = 400
GRID = N // BLK


def _sc_degree(col, wrep, zeros):
    chd = 80
    epw = E // (CORES * TILES)
    nch = epw // chd
    mesh = plsc.VectorSubcoreMesh(core_axis_name="c", subcore_axis_name="s")

    @functools.partial(
        pl.kernel,
        out_type=jax.ShapeDtypeStruct((2 * N, 16), jnp.float32),
        mesh=mesh,
        compiler_params=pltpu.CompilerParams(use_tc_tiling_on_sc=False),
        scratch_types=[
            pltpu.VMEM((chd,), jnp.int32),
            pltpu.VMEM((chd, 16), jnp.float32),
            pltpu.VMEM_SHARED((N, 16), jnp.float32),
        ],
    )
    def k(col_h, w_h, z_h, out_h, coli0, wv0, acc):
        c = lax.axis_index("c")
        s = lax.axis_index("s")
        rbase = jnp.minimum(s * RPT, RLAST)
        pltpu.sync_copy(z_h.at[pl.ds(rbase, RPT)], acc.at[pl.ds(rbase, RPT)])
        plsc.subcore_barrier()
        ebase = (c * TILES + s) * epw

        def body(i, carry):
            off = ebase + i * chd
            pltpu.sync_copy(col_h.at[pl.ds(off, chd)], coli0)
            pltpu.sync_copy(w_h.at[pl.ds(off, chd)], wv0)
            pltpu.sync_copy(wv0, acc.at[coli0], add=True)
            return carry

        lax.fori_loop(0, nch, body, 0)
        plsc.subcore_barrier()
        pltpu.sync_copy(acc.at[pl.ds(rbase, RPT)],
                        out_h.at[pl.ds(c * N + rbase, RPT)])

    return k(col, wrep, zeros)


def _sc_scatter(src, rowpc, col, wrep, dh):
    ept = E // TILES
    nch = ept // CH
    mesh = plsc.VectorSubcoreMesh(core_axis_name="c", subcore_axis_name="s")

    @functools.partial(
        pl.kernel,
        out_type=jax.ShapeDtypeStruct((2 * N, dh), jnp.float32),
        mesh=mesh,
        compiler_params=pltpu.CompilerParams(use_tc_tiling_on_sc=False),
        scratch_types=[
            pltpu.VMEM((CH,), jnp.int32),
            pltpu.VMEM((CH,), jnp.int32),
            pltpu.VMEM((CH, 16), jnp.float32),
            pltpu.VMEM((CH, dh), jnp.float32),
            pltpu.VMEM_SHARED((N, dh), jnp.float32),
            pltpu.SemaphoreType.DMA,
        ],
    )
    def k(src_h, row_h, col_h, w_h, out_h,
          rowi0, coli0, wv0, rows0, acc, sg0):
        c = lax.axis_index("c")
        s = lax.axis_index("s")
        rbase = jnp.minimum(s * RPT, RLAST)
        cn = c * N
        pltpu.sync_copy(src_h.at[pl.ds(cn + rbase, RPT)],
                        acc.at[pl.ds(rbase, RPT)])
        plsc.subcore_barrier()
        ebase = s * ept
        ce = c * E

        def process(rowi, coli, wv, rows, sem):
            pltpu.async_copy(src_h.at[rowi], rows, sem).wait()

            def scale(r, cy):
                wb = wv[r, pl.ds(0, 16)]
                for j in range(dh // 16):
                    rows[r, pl.ds(16 * j, 16)] = rows[r, pl.ds(16 * j, 16)] * wb
                return cy

            lax.fori_loop(0, CH, scale, 0)
            pltpu.sync_copy(rows, acc.at[coli], add=True)

        def body(i, carry):
            off = ebase + i * CH
            pltpu.sync_copy(row_h.at[pl.ds(ce + off, CH)], rowi0)
            pltpu.sync_copy(col_h.at[pl.ds(off, CH)], coli0)
            pltpu.sync_copy(w_h.at[pl.ds(off, CH)], wv0)
            process(rowi0, coli0, wv0, rows0, sg0)
            return carry

        lax.fori_loop(0, nch, body, 0)
        plsc.subcore_barrier()
        pltpu.sync_copy(acc.at[pl.ds(rbase, RPT)],
                        out_h.at[pl.ds(cn + rbase, RPT)])

    return k(src, rowpc, col, wrep)


def _tc_prep(x, degp):
    def body(x_ref, d0_ref, d1_ref, xs_ref, dis_ref):
        deg = d0_ref[:, 0:1] + d1_ref[:, 0:1] + 1.0
        dis = lax.rsqrt(deg)
        xs = x_ref[...] * dis
        xs_ref[0] = xs[:, : D_IN // 2]
        xs_ref[1] = xs[:, D_IN // 2:]
        dis_ref[...] = dis

    return pl.pallas_call(
        body,
        grid=(GRID,),
        in_specs=[
            pl.BlockSpec((BLK, D_IN), lambda i: (i, 0)),
            pl.BlockSpec((BLK, 16), lambda i: (i, 0)),
            pl.BlockSpec((BLK, 16), lambda i: (i, 0)),
        ],
        out_specs=[
            pl.BlockSpec((2, BLK, D_IN // 2), lambda i: (0, i, 0)),
            pl.BlockSpec((BLK, 1), lambda i: (i, 0)),
        ],
        out_shape=[
            jax.ShapeDtypeStruct((2, N, D_IN // 2), jnp.float32),
            jax.ShapeDtypeStruct((N, 1), jnp.float32),
        ],
    )(x, degp[:N], degp[N:])


def _tc_mm1(s1, dis, W1, b1):
    def body(s1_ref, dis_ref, w_ref, b_ref, y_ref):
        a = jnp.concatenate([s1_ref[0], s1_ref[1]], axis=1) * dis_ref[...]
        y_ref[...] = jnp.dot(a, w_ref[...],
                             preferred_element_type=jnp.float32) + b_ref[...]

    return pl.pallas_call(
        body,
        grid=(GRID,),
        in_specs=[
            pl.BlockSpec((2, BLK, D_IN // 2), lambda i: (0, i, 0)),
            pl.BlockSpec((BLK, 1), lambda i: (i, 0)),
            pl.BlockSpec((D_IN, H1), lambda i: (0, 0)),
            pl.BlockSpec((1, H1), lambda i: (0, 0)),
        ],
        out_specs=pl.BlockSpec((BLK, H1), lambda i: (i, 0)),
        out_shape=jax.ShapeDtypeStruct((N, H1), jnp.float32),
    )(s1, dis, W1, b1.reshape(1, H1))


def _tc_stats(y, h):
    def body(y_ref, mean_ref, rstd_ref, acc_ref):
        i = pl.program_id(0)

        @pl.when(i == 0)
        def _():
            acc_ref[...] = jnp.zeros_like(acc_ref)

        blk = y_ref[...]
        acc_ref[0:1] += jnp.sum(blk, axis=0, keepdims=True)
        acc_ref[1:2] += jnp.sum(blk * blk, axis=0, keepdims=True)

        @pl.when(i == GRID - 1)
        def _():
            m = acc_ref[0:1] / N
            v = acc_ref[1:2] / N - m * m
            mean_ref[...] = m
            rstd_ref[...] = lax.rsqrt(v + EPS)

    return pl.pallas_call(
        body,
        grid=(GRID,),
        in_specs=[pl.BlockSpec((BLK, h), lambda i: (i, 0))],
        out_specs=[
            pl.BlockSpec((1, h), lambda i: (0, 0)),
            pl.BlockSpec((1, h), lambda i: (0, 0)),
        ],
        out_shape=[jax.ShapeDtypeStruct((1, h), jnp.float32)] * 2,
        scratch_shapes=[pltpu.VMEM((2, h), jnp.float32)],
    )(y)


def _tc_mm2(y, mean, rstd, g1, be1, W2p, dis):
    def body(y_ref, m_ref, r_ref, g_ref, be_ref, w_ref, dis_ref, ts_ref):
        hh = jnp.maximum(
            g_ref[...] * (y_ref[...] - m_ref[...]) * r_ref[...] + be_ref[...],
            0.0)
        t = jnp.dot(hh, w_ref[...],
                    preferred_element_type=jnp.float32) * dis_ref[...]
        ts_ref[0] = t[:, : H2P // 2]
        ts_ref[1] = t[:, H2P // 2:]

    return pl.pallas_call(
        body,
        grid=(GRID,),
        in_specs=[
            pl.BlockSpec((BLK, H1), lambda i: (i, 0)),
            pl.BlockSpec((1, H1), lambda i: (0, 0)),
            pl.BlockSpec((1, H1), lambda i: (0, 0)),
            pl.BlockSpec((1, H1), lambda i: (0, 0)),
            pl.BlockSpec((1, H1), lambda i: (0, 0)),
            pl.BlockSpec((H1, H2P), lambda i: (0, 0)),
            pl.BlockSpec((BLK, 1), lambda i: (i, 0)),
        ],
        out_specs=pl.BlockSpec((2, BLK, H2P // 2), lambda i: (0, i, 0)),
        out_shape=jax.ShapeDtypeStruct((2, N, H2P // 2), jnp.float32),
    )(y, mean, rstd, g1.reshape(1, H1), be1.reshape(1, H1), W2p, dis)


def _tc_z(s2, dis, b2p):
    def body(s2_ref, dis_ref, b_ref, z_ref):
        z_ref[...] = (jnp.concatenate([s2_ref[0], s2_ref[1]], axis=1)
                      * dis_ref[...] + b_ref[...])

    return pl.pallas_call(
        body,
        grid=(GRID,),
        in_specs=[
            pl.BlockSpec((2, BLK, H2P // 2), lambda i: (0, i, 0)),
            pl.BlockSpec((BLK, 1), lambda i: (i, 0)),
            pl.BlockSpec((1, H2P), lambda i: (0, 0)),
        ],
        out_specs=pl.BlockSpec((BLK, H2P), lambda i: (i, 0)),
        out_shape=jax.ShapeDtypeStruct((N, H2P), jnp.float32),
    )(s2, dis, b2p)


def _tc_out(z, mean, rstd, g2p, be2p, Wlp, bl):
    def body(z_ref, m_ref, r_ref, g_ref, be_ref, w_ref, bl_ref, o_ref):
        hh = jnp.maximum(
            g_ref[...] * (z_ref[...] - m_ref[...]) * r_ref[...] + be_ref[...],
            0.0)
        lg = jnp.maximum(
            jnp.dot(hh, w_ref[...], preferred_element_type=jnp.float32)
            + bl_ref[...], 0.0)
        mx = jnp.max(lg, axis=1, keepdims=True)
        lse = jnp.log(jnp.sum(jnp.exp(lg - mx), axis=1, keepdims=True)) + mx
        o_ref[...] = lg - lse

    return pl.pallas_call(
        body,
        grid=(GRID,),
        in_specs=[
            pl.BlockSpec((BLK, H2P), lambda i: (i, 0)),
            pl.BlockSpec((1, H2P), lambda i: (0, 0)),
            pl.BlockSpec((1, H2P), lambda i: (0, 0)),
            pl.BlockSpec((1, H2P), lambda i: (0, 0)),
            pl.BlockSpec((1, H2P), lambda i: (0, 0)),
            pl.BlockSpec((H2P, D_OUT), lambda i: (0, 0)),
            pl.BlockSpec((1, D_OUT), lambda i: (0, 0)),
        ],
        out_specs=pl.BlockSpec((BLK, D_OUT), lambda i: (i, 0)),
        out_shape=jax.ShapeDtypeStruct((N, D_OUT), jnp.float32),
    )(z, mean, rstd, g2p, be2p, Wlp, bl.reshape(1, D_OUT))


def kernel(x, edge_index, edge_weight, W1, b1, g1, be1, W2, b2, g2, be2,
           Wl, bl):
    ei = edge_index.astype(jnp.int32)
    row = ei[0]
    col = ei[1]
    w = edge_weight.astype(jnp.float32)

    rowpc = jnp.concatenate([row, row + N])
    wrep = jnp.broadcast_to(w[:, None], (E, 16))

    degp = _sc_degree(col, wrep, jnp.zeros((N, 16), jnp.float32))
    xs, dis = _tc_prep(x, degp)

    s1 = _sc_scatter(xs.reshape(2 * N, D_IN // 2), rowpc, col, wrep,
                     D_IN // 2)
    y = _tc_mm1(s1.reshape(2, N, D_IN // 2), dis, W1, b1)
    m1, r1 = _tc_stats(y, H1)

    W2p = jnp.pad(W2, ((0, 0), (0, H2P - H2)))
    ts = _tc_mm2(y, m1, r1, g1, be1, W2p, dis)

    s2 = _sc_scatter(ts.reshape(2 * N, H2P // 2), rowpc, col, wrep, H2P // 2)
    b2p = jnp.pad(b2, (0, H2P - H2)).reshape(1, H2P)
    z = _tc_z(s2.reshape(2, N, H2P // 2), dis, b2p)
    m2, r2 = _tc_stats(z, H2P)

    g2p = jnp.pad(g2, (0, H2P - H2)).reshape(1, H2P)
    be2p = jnp.pad(be2, (0, H2P - H2)).reshape(1, H2P)
    Wlp = jnp.pad(Wl, ((0, H2P - H2), (0, 0)))
    return _tc_out(z, m2, r2, g2p, be2p, Wlp, bl)


# speedup vs baseline: 9.6550x; 1.5750x over previous
import functools

import jax
import jax.numpy as jnp
from jax import lax
from jax.experimental import pallas as pl
from jax.experimental.pallas import tpu as pltpu
from jax.experimental.pallas import tpu_sc as plsc

N = 10000
E = 320000
D_IN = 128
H1 = 512
H2 = 300
H2P = 320
D_OUT = 40
EPS = 1e-5

CH = 80
TILES = 16
CORES = 2
RPT = 640
RLAST = N - RPT

BLK = 400
GRID = N // BLK


def _sc_degree(col, wrep, zeros):
    chd = 80
    epw = E // (CORES * TILES)
    nch = epw // chd
    mesh = plsc.VectorSubcoreMesh(core_axis_name="c", subcore_axis_name="s")

    @functools.partial(
        pl.kernel,
        out_type=jax.ShapeDtypeStruct((2 * N, 16), jnp.float32),
        mesh=mesh,
        compiler_params=pltpu.CompilerParams(use_tc_tiling_on_sc=False),
        scratch_types=[
            pltpu.VMEM((chd,), jnp.int32),
            pltpu.VMEM((chd, 16), jnp.float32),
            pltpu.VMEM_SHARED((N, 16), jnp.float32),
        ],
    )
    def k(col_h, w_h, z_h, out_h, coli0, wv0, acc):
        c = lax.axis_index("c")
        s = lax.axis_index("s")
        rbase = jnp.minimum(s * RPT, RLAST)
        pltpu.sync_copy(z_h.at[pl.ds(rbase, RPT)], acc.at[pl.ds(rbase, RPT)])
        plsc.subcore_barrier()
        ebase = (c * TILES + s) * epw

        def body(i, carry):
            off = ebase + i * chd
            pltpu.sync_copy(col_h.at[pl.ds(off, chd)], coli0)
            pltpu.sync_copy(w_h.at[pl.ds(off, chd)], wv0)
            pltpu.sync_copy(wv0, acc.at[coli0], add=True)
            return carry

        lax.fori_loop(0, nch, body, 0)
        plsc.subcore_barrier()
        pltpu.sync_copy(acc.at[pl.ds(rbase, RPT)],
                        out_h.at[pl.ds(c * N + rbase, RPT)])

    return k(col, wrep, zeros)


def _sc_scatter(src, rowpc, col, wrep, dh):
    ept = E // TILES
    nch = ept // CH
    mesh = plsc.VectorSubcoreMesh(core_axis_name="c", subcore_axis_name="s")

    @functools.partial(
        pl.kernel,
        out_type=jax.ShapeDtypeStruct((2 * N, dh), jnp.float32),
        mesh=mesh,
        compiler_params=pltpu.CompilerParams(use_tc_tiling_on_sc=False),
        scratch_types=[
            pltpu.VMEM((CH,), jnp.int32),
            pltpu.VMEM((CH,), jnp.int32),
            pltpu.VMEM((CH, 16), jnp.float32),
            pltpu.VMEM((CH, dh), jnp.float32),
            pltpu.VMEM((CH,), jnp.int32),
            pltpu.VMEM((CH,), jnp.int32),
            pltpu.VMEM((CH, 16), jnp.float32),
            pltpu.VMEM((CH, dh), jnp.float32),
            pltpu.VMEM_SHARED((N, dh), jnp.float32),
            pltpu.SemaphoreType.DMA,
            pltpu.SemaphoreType.DMA,
            pltpu.SemaphoreType.DMA,
            pltpu.SemaphoreType.DMA,
        ],
    )
    def k(src_h, row_h, col_h, w_h, out_h,
          rowi0, coli0, wv0, rows0, rowi1, coli1, wv1, rows1,
          acc, si0, si1, sg0, sg1):
        c = lax.axis_index("c")
        s = lax.axis_index("s")
        rbase = jnp.minimum(s * RPT, RLAST)
        cn = c * N
        pltpu.sync_copy(src_h.at[pl.ds(cn + rbase, RPT)],
                        acc.at[pl.ds(rbase, RPT)])
        plsc.subcore_barrier()
        ebase = s * ept
        ce = c * E

        def issue_idx(i, rowi, coli, wv, sem):
            off = ebase + i * CH
            pltpu.async_copy(row_h.at[pl.ds(ce + off, CH)], rowi, sem)
            pltpu.async_copy(col_h.at[pl.ds(off, CH)], coli, sem)
            pltpu.async_copy(w_h.at[pl.ds(off, CH)], wv, sem)

        def wait_idx(rowi, coli, wv, sem):
            pltpu.make_async_copy(row_h.at[pl.ds(0, CH)], rowi, sem).wait()
            pltpu.make_async_copy(col_h.at[pl.ds(0, CH)], coli, sem).wait()
            pltpu.make_async_copy(w_h.at[pl.ds(0, CH)], wv, sem).wait()

        def process(rowi, coli, wv, rows, sem):
            pltpu.make_async_copy(src_h.at[rowi], rows, sem).wait()

            def scale(r, cy):
                wb = wv[r, pl.ds(0, 16)]
                for j in range(dh // 16):
                    rows[r, pl.ds(16 * j, 16)] = rows[r, pl.ds(16 * j, 16)] * wb
                return cy

            lax.fori_loop(0, CH, scale, 0)
            pltpu.sync_copy(rows, acc.at[coli], add=True)

        issue_idx(0, rowi0, coli0, wv0, si0)
        wait_idx(rowi0, coli0, wv0, si0)
        pltpu.async_copy(src_h.at[rowi0], rows0, sg0)
        issue_idx(1, rowi1, coli1, wv1, si1)

        def body(g, carry):
            i = 2 * g
            wait_idx(rowi1, coli1, wv1, si1)
            pltpu.async_copy(src_h.at[rowi1], rows1, sg1)
            process(rowi0, coli0, wv0, rows0, sg0)

            @pl.when(i + 2 < nch)
            def _():
                issue_idx(i + 2, rowi0, coli0, wv0, si0)
                wait_idx(rowi0, coli0, wv0, si0)
                pltpu.async_copy(src_h.at[rowi0], rows0, sg0)

            process(rowi1, coli1, wv1, rows1, sg1)

            @pl.when(i + 3 < nch)
            def _():
                issue_idx(i + 3, rowi1, coli1, wv1, si1)

            return carry

        lax.fori_loop(0, nch // 2, body, 0)
        plsc.subcore_barrier()
        pltpu.sync_copy(acc.at[pl.ds(rbase, RPT)],
                        out_h.at[pl.ds(cn + rbase, RPT)])

    return k(src, rowpc, col, wrep)


def _tc_prep(x, degp):
    def body(x_ref, d0_ref, d1_ref, xs_ref, dis_ref):
        deg = d0_ref[:, 0:1] + d1_ref[:, 0:1] + 1.0
        dis = lax.rsqrt(deg)
        xs = x_ref[...] * dis
        xs_ref[0] = xs[:, : D_IN // 2]
        xs_ref[1] = xs[:, D_IN // 2:]
        dis_ref[...] = dis

    return pl.pallas_call(
        body,
        grid=(GRID,),
        in_specs=[
            pl.BlockSpec((BLK, D_IN), lambda i: (i, 0)),
            pl.BlockSpec((BLK, 16), lambda i: (i, 0)),
            pl.BlockSpec((BLK, 16), lambda i: (i, 0)),
        ],
        out_specs=[
            pl.BlockSpec((2, BLK, D_IN // 2), lambda i: (0, i, 0)),
            pl.BlockSpec((BLK, 1), lambda i: (i, 0)),
        ],
        out_shape=[
            jax.ShapeDtypeStruct((2, N, D_IN // 2), jnp.float32),
            jax.ShapeDtypeStruct((N, 1), jnp.float32),
        ],
    )(x, degp[:N], degp[N:])


def _tc_mm1(s1, dis, W1, b1):
    def body(s1_ref, dis_ref, w_ref, b_ref, y_ref):
        a = jnp.concatenate([s1_ref[0], s1_ref[1]], axis=1) * dis_ref[...]
        y_ref[...] = jnp.dot(a, w_ref[...],
                             preferred_element_type=jnp.float32) + b_ref[...]

    return pl.pallas_call(
        body,
        grid=(GRID,),
        in_specs=[
            pl.BlockSpec((2, BLK, D_IN // 2), lambda i: (0, i, 0)),
            pl.BlockSpec((BLK, 1), lambda i: (i, 0)),
            pl.BlockSpec((D_IN, H1), lambda i: (0, 0)),
            pl.BlockSpec((1, H1), lambda i: (0, 0)),
        ],
        out_specs=pl.BlockSpec((BLK, H1), lambda i: (i, 0)),
        out_shape=jax.ShapeDtypeStruct((N, H1), jnp.float32),
    )(s1, dis, W1, b1.reshape(1, H1))


def _tc_stats(y, h):
    def body(y_ref, mean_ref, rstd_ref, acc_ref):
        i = pl.program_id(0)

        @pl.when(i == 0)
        def _():
            acc_ref[...] = jnp.zeros_like(acc_ref)

        blk = y_ref[...]
        acc_ref[0:1] += jnp.sum(blk, axis=0, keepdims=True)
        acc_ref[1:2] += jnp.sum(blk * blk, axis=0, keepdims=True)

        @pl.when(i == GRID - 1)
        def _():
            m = acc_ref[0:1] / N
            v = acc_ref[1:2] / N - m * m
            mean_ref[...] = m
            rstd_ref[...] = lax.rsqrt(v + EPS)

    return pl.pallas_call(
        body,
        grid=(GRID,),
        in_specs=[pl.BlockSpec((BLK, h), lambda i: (i, 0))],
        out_specs=[
            pl.BlockSpec((1, h), lambda i: (0, 0)),
            pl.BlockSpec((1, h), lambda i: (0, 0)),
        ],
        out_shape=[jax.ShapeDtypeStruct((1, h), jnp.float32)] * 2,
        scratch_shapes=[pltpu.VMEM((2, h), jnp.float32)],
    )(y)


def _tc_mm2(y, mean, rstd, g1, be1, W2p, dis):
    def body(y_ref, m_ref, r_ref, g_ref, be_ref, w_ref, dis_ref, ts_ref):
        hh = jnp.maximum(
            g_ref[...] * (y_ref[...] - m_ref[...]) * r_ref[...] + be_ref[...],
            0.0)
        t = jnp.dot(hh, w_ref[...],
                    preferred_element_type=jnp.float32) * dis_ref[...]
        ts_ref[0] = t[:, : H2P // 2]
        ts_ref[1] = t[:, H2P // 2:]

    return pl.pallas_call(
        body,
        grid=(GRID,),
        in_specs=[
            pl.BlockSpec((BLK, H1), lambda i: (i, 0)),
            pl.BlockSpec((1, H1), lambda i: (0, 0)),
            pl.BlockSpec((1, H1), lambda i: (0, 0)),
            pl.BlockSpec((1, H1), lambda i: (0, 0)),
            pl.BlockSpec((1, H1), lambda i: (0, 0)),
            pl.BlockSpec((H1, H2P), lambda i: (0, 0)),
            pl.BlockSpec((BLK, 1), lambda i: (i, 0)),
        ],
        out_specs=pl.BlockSpec((2, BLK, H2P // 2), lambda i: (0, i, 0)),
        out_shape=jax.ShapeDtypeStruct((2, N, H2P // 2), jnp.float32),
    )(y, mean, rstd, g1.reshape(1, H1), be1.reshape(1, H1), W2p, dis)


def _tc_z(s2, dis, b2p):
    def body(s2_ref, dis_ref, b_ref, z_ref):
        z_ref[...] = (jnp.concatenate([s2_ref[0], s2_ref[1]], axis=1)
                      * dis_ref[...] + b_ref[...])

    return pl.pallas_call(
        body,
        grid=(GRID,),
        in_specs=[
            pl.BlockSpec((2, BLK, H2P // 2), lambda i: (0, i, 0)),
            pl.BlockSpec((BLK, 1), lambda i: (i, 0)),
            pl.BlockSpec((1, H2P), lambda i: (0, 0)),
        ],
        out_specs=pl.BlockSpec((BLK, H2P), lambda i: (i, 0)),
        out_shape=jax.ShapeDtypeStruct((N, H2P), jnp.float32),
    )(s2, dis, b2p)


def _tc_out(z, mean, rstd, g2p, be2p, Wlp, bl):
    def body(z_ref, m_ref, r_ref, g_ref, be_ref, w_ref, bl_ref, o_ref):
        hh = jnp.maximum(
            g_ref[...] * (z_ref[...] - m_ref[...]) * r_ref[...] + be_ref[...],
            0.0)
        lg = jnp.maximum(
            jnp.dot(hh, w_ref[...], preferred_element_type=jnp.float32)
            + bl_ref[...], 0.0)
        mx = jnp.max(lg, axis=1, keepdims=True)
        lse = jnp.log(jnp.sum(jnp.exp(lg - mx), axis=1, keepdims=True)) + mx
        o_ref[...] = lg - lse

    return pl.pallas_call(
        body,
        grid=(GRID,),
        in_specs=[
            pl.BlockSpec((BLK, H2P), lambda i: (i, 0)),
            pl.BlockSpec((1, H2P), lambda i: (0, 0)),
            pl.BlockSpec((1, H2P), lambda i: (0, 0)),
            pl.BlockSpec((1, H2P), lambda i: (0, 0)),
            pl.BlockSpec((1, H2P), lambda i: (0, 0)),
            pl.BlockSpec((H2P, D_OUT), lambda i: (0, 0)),
            pl.BlockSpec((1, D_OUT), lambda i: (0, 0)),
        ],
        out_specs=pl.BlockSpec((BLK, D_OUT), lambda i: (i, 0)),
        out_shape=jax.ShapeDtypeStruct((N, D_OUT), jnp.float32),
    )(z, mean, rstd, g2p, be2p, Wlp, bl.reshape(1, D_OUT))


def kernel(x, edge_index, edge_weight, W1, b1, g1, be1, W2, b2, g2, be2,
           Wl, bl):
    ei = edge_index.astype(jnp.int32)
    row = ei[0]
    col = ei[1]
    w = edge_weight.astype(jnp.float32)

    rowpc = jnp.concatenate([row, row + N])
    wrep = jnp.broadcast_to(w[:, None], (E, 16))

    degp = _sc_degree(col, wrep, jnp.zeros((N, 16), jnp.float32))
    xs, dis = _tc_prep(x, degp)

    s1 = _sc_scatter(xs.reshape(2 * N, D_IN // 2), rowpc, col, wrep,
                     D_IN // 2)
    y = _tc_mm1(s1.reshape(2, N, D_IN // 2), dis, W1, b1)
    m1, r1 = _tc_stats(y, H1)

    W2p = jnp.pad(W2, ((0, 0), (0, H2P - H2)))
    ts = _tc_mm2(y, m1, r1, g1, be1, W2p, dis)

    s2 = _sc_scatter(ts.reshape(2 * N, H2P // 2), rowpc, col, wrep, H2P // 2)
    b2p = jnp.pad(b2, (0, H2P - H2)).reshape(1, H2P)
    z = _tc_z(s2.reshape(2, N, H2P // 2), dis, b2p)
    m2, r2 = _tc_stats(z, H2P)

    g2p = jnp.pad(g2, (0, H2P - H2)).reshape(1, H2P)
    be2p = jnp.pad(be2, (0, H2P - H2)).reshape(1, H2P)
    Wlp = jnp.pad(Wl, ((0, H2P - H2), (0, 0)))
    return _tc_out(z, m2, r2, g2p, be2p, Wlp, bl)


# speedup vs baseline: 10.0513x; 1.0410x over previous
import functools

import jax
import jax.numpy as jnp
from jax import lax
from jax.experimental import pallas as pl
from jax.experimental.pallas import tpu as pltpu
from jax.experimental.pallas import tpu_sc as plsc

N = 10000
E = 320000
D_IN = 128
H1 = 512
H2 = 300
H2P = 320
D_OUT = 40
EPS = 1e-5

CH = 80
TILES = 16
CORES = 2
RPT = 640
RLAST = N - RPT

BLK = 400
GRID = N // BLK


def _sc_degree(col, wrep, zeros):
    chd = 40
    epw = E // (CORES * TILES)
    nch = epw // chd
    mesh = plsc.VectorSubcoreMesh(core_axis_name="c", subcore_axis_name="s")

    @functools.partial(
        pl.kernel,
        out_type=jax.ShapeDtypeStruct((2 * N, 16), jnp.float32),
        mesh=mesh,
        compiler_params=pltpu.CompilerParams(use_tc_tiling_on_sc=False),
        scratch_types=[
            pltpu.VMEM((chd,), jnp.int32),
            pltpu.VMEM((chd, 16), jnp.float32),
            pltpu.VMEM((chd,), jnp.int32),
            pltpu.VMEM((chd, 16), jnp.float32),
            pltpu.VMEM_SHARED((N, 16), jnp.float32),
            pltpu.SemaphoreType.DMA,
            pltpu.SemaphoreType.DMA,
        ],
    )
    def k(col_h, w_h, z_h, out_h, coli0, wv0, coli1, wv1, acc, si0, si1):
        c = lax.axis_index("c")
        s = lax.axis_index("s")
        rbase = jnp.minimum(s * RPT, RLAST)
        pltpu.sync_copy(z_h.at[pl.ds(rbase, RPT)], acc.at[pl.ds(rbase, RPT)])
        plsc.subcore_barrier()
        ebase = (c * TILES + s) * epw

        def issue(i, coli, wv, sem):
            off = ebase + i * chd
            pltpu.async_copy(col_h.at[pl.ds(off, chd)], coli, sem)
            pltpu.async_copy(w_h.at[pl.ds(off, chd)], wv, sem)

        def wait(coli, wv, sem):
            pltpu.make_async_copy(col_h.at[pl.ds(0, chd)], coli, sem).wait()
            pltpu.make_async_copy(w_h.at[pl.ds(0, chd)], wv, sem).wait()

        issue(0, coli0, wv0, si0)

        def body(g, carry):
            i = 2 * g
            issue(i + 1, coli1, wv1, si1)
            wait(coli0, wv0, si0)
            pltpu.sync_copy(wv0, acc.at[coli0], add=True)

            @pl.when(i + 2 < nch)
            def _():
                issue(i + 2, coli0, wv0, si0)

            wait(coli1, wv1, si1)
            pltpu.sync_copy(wv1, acc.at[coli1], add=True)
            return carry

        lax.fori_loop(0, nch // 2, body, 0)
        plsc.subcore_barrier()
        pltpu.sync_copy(acc.at[pl.ds(rbase, RPT)],
                        out_h.at[pl.ds(c * N + rbase, RPT)])

    return k(col, wrep, zeros)


def _sc_scatter(src, rowpc, col, wrep, dh):
    ept = E // TILES
    nch = ept // CH
    mesh = plsc.VectorSubcoreMesh(core_axis_name="c", subcore_axis_name="s")

    @functools.partial(
        pl.kernel,
        out_type=jax.ShapeDtypeStruct((2 * N, dh), jnp.float32),
        mesh=mesh,
        compiler_params=pltpu.CompilerParams(use_tc_tiling_on_sc=False),
        scratch_types=[
            pltpu.VMEM((CH,), jnp.int32),
            pltpu.VMEM((CH,), jnp.int32),
            pltpu.VMEM((CH, 16), jnp.float32),
            pltpu.VMEM((CH, dh), jnp.float32),
            pltpu.VMEM((CH,), jnp.int32),
            pltpu.VMEM((CH,), jnp.int32),
            pltpu.VMEM((CH, 16), jnp.float32),
            pltpu.VMEM((CH, dh), jnp.float32),
            pltpu.VMEM_SHARED((N, dh), jnp.float32),
            pltpu.SemaphoreType.DMA,
            pltpu.SemaphoreType.DMA,
            pltpu.SemaphoreType.DMA,
            pltpu.SemaphoreType.DMA,
        ],
    )
    def k(src_h, row_h, col_h, w_h, out_h,
          rowi0, coli0, wv0, rows0, rowi1, coli1, wv1, rows1,
          acc, si0, si1, sg0, sg1):
        c = lax.axis_index("c")
        s = lax.axis_index("s")
        rbase = jnp.minimum(s * RPT, RLAST)
        cn = c * N
        pltpu.sync_copy(src_h.at[pl.ds(cn + rbase, RPT)],
                        acc.at[pl.ds(rbase, RPT)])
        plsc.subcore_barrier()
        ebase = s * ept
        ce = c * E

        def issue_idx(i, rowi, coli, wv, sem):
            off = ebase + i * CH
            pltpu.async_copy(row_h.at[pl.ds(ce + off, CH)], rowi, sem)
            pltpu.async_copy(col_h.at[pl.ds(off, CH)], coli, sem)
            pltpu.async_copy(w_h.at[pl.ds(off, CH)], wv, sem)

        def wait_idx(rowi, coli, wv, sem):
            pltpu.make_async_copy(row_h.at[pl.ds(0, CH)], rowi, sem).wait()
            pltpu.make_async_copy(col_h.at[pl.ds(0, CH)], coli, sem).wait()
            pltpu.make_async_copy(w_h.at[pl.ds(0, CH)], wv, sem).wait()

        def process(rowi, coli, wv, rows, sem):
            pltpu.make_async_copy(src_h.at[rowi], rows, sem).wait()

            def scale(r, cy):
                wb = wv[r, pl.ds(0, 16)]
                for j in range(dh // 16):
                    rows[r, pl.ds(16 * j, 16)] = rows[r, pl.ds(16 * j, 16)] * wb
                return cy

            lax.fori_loop(0, CH, scale, 0)
            pltpu.sync_copy(rows, acc.at[coli], add=True)

        issue_idx(0, rowi0, coli0, wv0, si0)
        wait_idx(rowi0, coli0, wv0, si0)
        pltpu.async_copy(src_h.at[rowi0], rows0, sg0)
        issue_idx(1, rowi1, coli1, wv1, si1)

        def body(g, carry):
            i = 2 * g
            wait_idx(rowi1, coli1, wv1, si1)
            pltpu.async_copy(src_h.at[rowi1], rows1, sg1)
            process(rowi0, coli0, wv0, rows0, sg0)

            @pl.when(i + 2 < nch)
            def _():
                issue_idx(i + 2, rowi0, coli0, wv0, si0)
                wait_idx(rowi0, coli0, wv0, si0)
                pltpu.async_copy(src_h.at[rowi0], rows0, sg0)

            process(rowi1, coli1, wv1, rows1, sg1)

            @pl.when(i + 3 < nch)
            def _():
                issue_idx(i + 3, rowi1, coli1, wv1, si1)

            return carry

        lax.fori_loop(0, nch // 2, body, 0)
        plsc.subcore_barrier()
        pltpu.sync_copy(acc.at[pl.ds(rbase, RPT)],
                        out_h.at[pl.ds(cn + rbase, RPT)])

    return k(src, rowpc, col, wrep)


def _tc_prep(x, degp):
    def body(x_ref, d0_ref, d1_ref, xs_ref, dis_ref):
        deg = d0_ref[:, 0:1] + d1_ref[:, 0:1] + 1.0
        dis = lax.rsqrt(deg)
        xs = x_ref[...] * dis
        xs_ref[0] = xs[:, : D_IN // 2]
        xs_ref[1] = xs[:, D_IN // 2:]
        dis_ref[...] = dis

    return pl.pallas_call(
        body,
        grid=(GRID,),
        in_specs=[
            pl.BlockSpec((BLK, D_IN), lambda i: (i, 0)),
            pl.BlockSpec((BLK, 16), lambda i: (i, 0)),
            pl.BlockSpec((BLK, 16), lambda i: (i, 0)),
        ],
        out_specs=[
            pl.BlockSpec((2, BLK, D_IN // 2), lambda i: (0, i, 0)),
            pl.BlockSpec((BLK, 1), lambda i: (i, 0)),
        ],
        out_shape=[
            jax.ShapeDtypeStruct((2, N, D_IN // 2), jnp.float32),
            jax.ShapeDtypeStruct((N, 1), jnp.float32),
        ],
    )(x, degp[:N], degp[N:])


def _tc_mm1(s1, dis, W1, b1):
    def body(s1_ref, dis_ref, w_ref, b_ref, y_ref):
        a = jnp.concatenate([s1_ref[0], s1_ref[1]], axis=1) * dis_ref[...]
        y_ref[...] = jnp.dot(a, w_ref[...],
                             preferred_element_type=jnp.float32) + b_ref[...]

    return pl.pallas_call(
        body,
        grid=(GRID,),
        in_specs=[
            pl.BlockSpec((2, BLK, D_IN // 2), lambda i: (0, i, 0)),
            pl.BlockSpec((BLK, 1), lambda i: (i, 0)),
            pl.BlockSpec((D_IN, H1), lambda i: (0, 0)),
            pl.BlockSpec((1, H1), lambda i: (0, 0)),
        ],
        out_specs=pl.BlockSpec((BLK, H1), lambda i: (i, 0)),
        out_shape=jax.ShapeDtypeStruct((N, H1), jnp.float32),
    )(s1, dis, W1, b1.reshape(1, H1))


def _tc_stats(y, h):
    def body(y_ref, mean_ref, rstd_ref, acc_ref):
        i = pl.program_id(0)

        @pl.when(i == 0)
        def _():
            acc_ref[...] = jnp.zeros_like(acc_ref)

        blk = y_ref[...]
        acc_ref[0:1] += jnp.sum(blk, axis=0, keepdims=True)
        acc_ref[1:2] += jnp.sum(blk * blk, axis=0, keepdims=True)

        @pl.when(i == GRID - 1)
        def _():
            m = acc_ref[0:1] / N
            v = acc_ref[1:2] / N - m * m
            mean_ref[...] = m
            rstd_ref[...] = lax.rsqrt(v + EPS)

    return pl.pallas_call(
        body,
        grid=(GRID,),
        in_specs=[pl.BlockSpec((BLK, h), lambda i: (i, 0))],
        out_specs=[
            pl.BlockSpec((1, h), lambda i: (0, 0)),
            pl.BlockSpec((1, h), lambda i: (0, 0)),
        ],
        out_shape=[jax.ShapeDtypeStruct((1, h), jnp.float32)] * 2,
        scratch_shapes=[pltpu.VMEM((2, h), jnp.float32)],
    )(y)


def _tc_mm2(y, mean, rstd, g1, be1, W2p, dis):
    def body(y_ref, m_ref, r_ref, g_ref, be_ref, w_ref, dis_ref, ts_ref):
        hh = jnp.maximum(
            g_ref[...] * (y_ref[...] - m_ref[...]) * r_ref[...] + be_ref[...],
            0.0)
        t = jnp.dot(hh, w_ref[...],
                    preferred_element_type=jnp.float32) * dis_ref[...]
        ts_ref[0] = t[:, : H2P // 2]
        ts_ref[1] = t[:, H2P // 2:]

    return pl.pallas_call(
        body,
        grid=(GRID,),
        in_specs=[
            pl.BlockSpec((BLK, H1), lambda i: (i, 0)),
            pl.BlockSpec((1, H1), lambda i: (0, 0)),
            pl.BlockSpec((1, H1), lambda i: (0, 0)),
            pl.BlockSpec((1, H1), lambda i: (0, 0)),
            pl.BlockSpec((1, H1), lambda i: (0, 0)),
            pl.BlockSpec((H1, H2P), lambda i: (0, 0)),
            pl.BlockSpec((BLK, 1), lambda i: (i, 0)),
        ],
        out_specs=pl.BlockSpec((2, BLK, H2P // 2), lambda i: (0, i, 0)),
        out_shape=jax.ShapeDtypeStruct((2, N, H2P // 2), jnp.float32),
    )(y, mean, rstd, g1.reshape(1, H1), be1.reshape(1, H1), W2p, dis)


def _tc_z(s2, dis, b2p):
    def body(s2_ref, dis_ref, b_ref, z_ref):
        z_ref[...] = (jnp.concatenate([s2_ref[0], s2_ref[1]], axis=1)
                      * dis_ref[...] + b_ref[...])

    return pl.pallas_call(
        body,
        grid=(GRID,),
        in_specs=[
            pl.BlockSpec((2, BLK, H2P // 2), lambda i: (0, i, 0)),
            pl.BlockSpec((BLK, 1), lambda i: (i, 0)),
            pl.BlockSpec((1, H2P), lambda i: (0, 0)),
        ],
        out_specs=pl.BlockSpec((BLK, H2P), lambda i: (i, 0)),
        out_shape=jax.ShapeDtypeStruct((N, H2P), jnp.float32),
    )(s2, dis, b2p)


def _tc_out(z, mean, rstd, g2p, be2p, Wlp, bl):
    def body(z_ref, m_ref, r_ref, g_ref, be_ref, w_ref, bl_ref, o_ref):
        hh = jnp.maximum(
            g_ref[...] * (z_ref[...] - m_ref[...]) * r_ref[...] + be_ref[...],
            0.0)
        lg = jnp.maximum(
            jnp.dot(hh, w_ref[...], preferred_element_type=jnp.float32)
            + bl_ref[...], 0.0)
        mx = jnp.max(lg, axis=1, keepdims=True)
        lse = jnp.log(jnp.sum(jnp.exp(lg - mx), axis=1, keepdims=True)) + mx
        o_ref[...] = lg - lse

    return pl.pallas_call(
        body,
        grid=(GRID,),
        in_specs=[
            pl.BlockSpec((BLK, H2P), lambda i: (i, 0)),
            pl.BlockSpec((1, H2P), lambda i: (0, 0)),
            pl.BlockSpec((1, H2P), lambda i: (0, 0)),
            pl.BlockSpec((1, H2P), lambda i: (0, 0)),
            pl.BlockSpec((1, H2P), lambda i: (0, 0)),
            pl.BlockSpec((H2P, D_OUT), lambda i: (0, 0)),
            pl.BlockSpec((1, D_OUT), lambda i: (0, 0)),
        ],
        out_specs=pl.BlockSpec((BLK, D_OUT), lambda i: (i, 0)),
        out_shape=jax.ShapeDtypeStruct((N, D_OUT), jnp.float32),
    )(z, mean, rstd, g2p, be2p, Wlp, bl.reshape(1, D_OUT))


def kernel(x, edge_index, edge_weight, W1, b1, g1, be1, W2, b2, g2, be2,
           Wl, bl):
    ei = edge_index.astype(jnp.int32)
    row = ei[0]
    col = ei[1]
    w = edge_weight.astype(jnp.float32)

    rowpc = jnp.concatenate([row, row + N])
    wrep = jnp.broadcast_to(w[:, None], (E, 16))

    degp = _sc_degree(col, wrep, jnp.zeros((N, 16), jnp.float32))
    xs, dis = _tc_prep(x, degp)

    s1 = _sc_scatter(xs.reshape(2 * N, D_IN // 2), rowpc, col, wrep,
                     D_IN // 2)
    y = _tc_mm1(s1.reshape(2, N, D_IN // 2), dis, W1, b1)
    m1, r1 = _tc_stats(y, H1)

    W2p = jnp.pad(W2, ((0, 0), (0, H2P - H2)))
    ts = _tc_mm2(y, m1, r1, g1, be1, W2p, dis)

    s2 = _sc_scatter(ts.reshape(2 * N, H2P // 2), rowpc, col, wrep, H2P // 2)
    b2p = jnp.pad(b2, (0, H2P - H2)).reshape(1, H2P)
    z = _tc_z(s2.reshape(2, N, H2P // 2), dis, b2p)
    m2, r2 = _tc_stats(z, H2P)

    g2p = jnp.pad(g2, (0, H2P - H2)).reshape(1, H2P)
    be2p = jnp.pad(be2, (0, H2P - H2)).reshape(1, H2P)
    Wlp = jnp.pad(Wl, ((0, H2P - H2), (0, 0)))
    return _tc_out(z, m2, r2, g2p, be2p, Wlp, bl)
